# baseline (speedup 1.0000x reference)
import functools

import jax
import jax.numpy as jnp
from jax import lax
from jax.experimental import pallas as pl
from jax.experimental.pallas import tpu as pltpu
from jax.experimental.pallas import tpu_sc as plsc

_NUM_CORES = 2
_NUM_SUBCORES = 16
_NUM_WORKERS = _NUM_CORES * _NUM_SUBCORES
_CHUNK = 32
_NBUF = 2


def _make_lookup(batch: int, seq: int, vocab: int, d: int):
    n_ids = batch * seq
    assert n_ids % (_NUM_WORKERS * _CHUNK) == 0
    b_per_w = n_ids // _NUM_WORKERS
    assert seq % b_per_w == 0
    w_per_row = seq // b_per_w
    n_chunks = b_per_w // _CHUNK
    mesh = plsc.VectorSubcoreMesh(core_axis_name="c", subcore_axis_name="s")

    @functools.partial(
        pl.kernel,
        out_type=jax.ShapeDtypeStruct((batch, seq, d), jnp.float32),
        mesh=mesh,
        compiler_params=pltpu.CompilerParams(skip_device_barrier=True),
        scratch_types=[
            pltpu.VMEM((b_per_w,), jnp.int32),
            [pltpu.VMEM((_CHUNK, d), jnp.float32) for _ in range(_NBUF)],
            [pltpu.SemaphoreType.DMA for _ in range(_NBUF)],
            [pltpu.SemaphoreType.DMA for _ in range(_NBUF)],
        ],
    )
    def lookup(ids_hbm, table_hbm, out_hbm, idx_v, bufs, gsems, wsems):
        wid = lax.axis_index("s") * _NUM_CORES + lax.axis_index("c")
        row = wid // w_per_row
        col = (wid % w_per_row) * b_per_w
        pltpu.sync_copy(ids_hbm.at[row, pl.ds(col, b_per_w)], idx_v)

        max_off = (n_chunks - 1) * _CHUNK

        def start_gather(g, b):
            off = lax.min(g * _CHUNK, max_off)
            pltpu.async_copy(
                table_hbm.at[idx_v.at[pl.ds(off, _CHUNK)]], bufs[b], gsems[b]
            )

        def wait_gather(b):
            pltpu.make_async_copy(
                table_hbm.at[idx_v.at[pl.ds(0, _CHUNK)]], bufs[b], gsems[b]
            ).wait()

        def sync_write(g, b):
            pltpu.sync_copy(
                bufs[b], out_hbm.at[row, pl.ds(col + g * _CHUNK, _CHUNK)]
            )

        start_gather(0, 0)

        def pair_body(h, carry):
            g0 = 2 * h
            start_gather(g0 + 1, 1)
            wait_gather(0)
            sync_write(g0, 0)
            start_gather(g0 + 2, 0)
            wait_gather(1)
            sync_write(g0 + 1, 1)
            return carry

        lax.fori_loop(0, n_chunks // 2, pair_body, 0)
        wait_gather(0)

    return lookup


def kernel(input_ids, table):
    b, s = input_ids.shape
    vocab, d = table.shape
    return _make_lookup(b, s, vocab, d)(input_ids, table)


# speedup vs baseline: 1.0019x; 1.0019x over previous
import functools

import jax
import jax.numpy as jnp
from jax import lax
from jax.experimental import pallas as pl
from jax.experimental.pallas import tpu as pltpu
from jax.experimental.pallas import tpu_sc as plsc

_NUM_CORES = 2
_NUM_SUBCORES = 16
_NUM_WORKERS = _NUM_CORES * _NUM_SUBCORES
_CHUNK = 32
_NBUF = 2


def _make_lookup(batch: int, seq: int, vocab: int, d: int):
    n_ids = batch * seq
    assert n_ids % (_NUM_WORKERS * _CHUNK) == 0
    b_per_w = n_ids // _NUM_WORKERS
    assert seq % b_per_w == 0
    w_per_row = seq // b_per_w
    n_chunks = b_per_w // _CHUNK
    mesh = plsc.VectorSubcoreMesh(core_axis_name="c", subcore_axis_name="s")

    @functools.partial(
        pl.kernel,
        out_type=jax.ShapeDtypeStruct((batch, seq, d), jnp.float32),
        mesh=mesh,
        scratch_types=[
            pltpu.VMEM((b_per_w,), jnp.int32),
            [pltpu.VMEM((_CHUNK, d), jnp.float32) for _ in range(_NBUF)],
            [pltpu.SemaphoreType.DMA for _ in range(_NBUF)],
        ],
    )
    def lookup(ids_hbm, table_hbm, out_hbm, idx_v, bufs, gsems):
        wid = lax.axis_index("s") * _NUM_CORES + lax.axis_index("c")
        row = wid // w_per_row
        col = (wid % w_per_row) * b_per_w
        pltpu.sync_copy(ids_hbm.at[row, pl.ds(col, b_per_w)], idx_v)

        max_off = (n_chunks - 1) * _CHUNK

        def start_gather(g, b):
            off = lax.min(g * _CHUNK, max_off)
            pltpu.async_copy(
                table_hbm.at[idx_v.at[pl.ds(off, _CHUNK)]], bufs[b], gsems[b]
            )

        def wait_gather(b):
            pltpu.make_async_copy(
                table_hbm.at[idx_v.at[pl.ds(0, _CHUNK)]], bufs[b], gsems[b]
            ).wait()

        def sync_write(g, b):
            pltpu.sync_copy(
                bufs[b], out_hbm.at[row, pl.ds(col + g * _CHUNK, _CHUNK)]
            )

        start_gather(0, 0)

        def pair_body(h, carry):
            g0 = 2 * h
            start_gather(g0 + 1, 1)
            wait_gather(0)
            sync_write(g0, 0)
            start_gather(g0 + 2, 0)
            wait_gather(1)
            sync_write(g0 + 1, 1)
            return carry

        lax.fori_loop(0, n_chunks // 2, pair_body, 0)
        wait_gather(0)

    return lookup


def kernel(input_ids, table):
    b, s = input_ids.shape
    vocab, d = table.shape
    return _make_lookup(b, s, vocab, d)(input_ids, table)
